# baseline (speedup 1.0000x reference)
import functools

import jax
import jax.numpy as jnp
from jax import lax
from jax.experimental import pallas as pl
from jax.experimental.pallas import tpu as pltpu
from jax.experimental.pallas import tpu_sc as plsc

B, T, E = 32, 2048, 64
NC, NS, L = 2, 16, 16
NW = NC * NS
CHUNK = 256
NCHUNK = T // CHUNK
G = E // L


def _body(idx_hbm, tok_hbm, pos_hbm, out_hbm, idx_v, tok_v, pos_v, sem_in):
    cid = lax.axis_index("c")
    sid = lax.axis_index("s")
    w = sid * NC + cid
    pltpu.sync_copy(idx_hbm.at[pl.ds(w * T, T)], idx_v)
    for c in range(NCHUNK):
        slot = c % 2
        g = pltpu.async_copy(
            tok_hbm.at[idx_v.at[pl.ds(c * CHUNK, CHUNK)]], tok_v.at[slot], sem_in
        )
        p = pltpu.async_copy(
            pos_hbm.at[pl.ds(c * CHUNK, CHUNK)], pos_v.at[slot], sem_in
        )
        g.wait()
        p.wait()

        def row(r, carry, slot=slot):
            for k in range(G):
                pv = pos_v[slot, r, pl.ds(k * L, L)]
                plsc.addupdate(tok_v.at[slot, r, pl.ds(k * L, L)], pv)
            return carry

        lax.fori_loop(0, CHUNK, row, 0)
        pltpu.sync_copy(
            tok_v.at[slot], out_hbm.at[pl.ds(w * T + c * CHUNK, CHUNK)]
        )


@jax.jit
def _emb(idx_flat, token_table, pos_table):
    mesh = plsc.VectorSubcoreMesh(
        core_axis_name="c", subcore_axis_name="s", num_cores=NC, num_subcores=NS
    )
    f = pl.kernel(
        _body,
        out_type=jax.ShapeDtypeStruct((B * T, E), jnp.float32),
        mesh=mesh,
        scratch_types=[
            pltpu.VMEM((B * T // NW,), jnp.int32),
            pltpu.VMEM((2, CHUNK, E), jnp.float32),
            pltpu.VMEM((2, CHUNK, E), jnp.float32),
            pltpu.SemaphoreType.DMA,
        ],
        compiler_params=pltpu.CompilerParams(use_tc_tiling_on_sc=False),
    )
    return f(idx_flat, token_table, pos_table)


def kernel(idx, token_table, pos_table):
    out = _emb(idx.reshape(-1).astype(jnp.int32), token_table, pos_table)
    return out.reshape(B, T, E)
